# baseline (speedup 1.0000x reference)
import jax
import jax.numpy as jnp
from jax import lax
from jax.experimental import pallas as pl
from jax.experimental.pallas import tpu as pltpu, tpu_sc as plsc

_B = 16384
_D = 32
_CHUNK = 128
_NROW = 1000001
_STRIDE = 1000008


def _build_sc_call():
    info = plsc.get_sparse_core_info()
    nc, ns = info.num_cores, info.num_subcores
    nw = nc * ns
    bpw = _B // nw
    nchunk = bpw // _CHUNK

    mesh = plsc.VectorSubcoreMesh(core_axis_name="c", subcore_axis_name="s")

    def body(user_hbm, pos_hbm, neg_hbm, utab_hbm, itab_hbm,
             pos_out, neg_out,
             uidx_v, pidx_v, nidx_v, ubuf_v, pbuf_v, nbuf_v,
             posbuf_v, negbuf_v, sem):
        wid = lax.axis_index("s") * nc + lax.axis_index("c")
        ibase = wid * nchunk

        pltpu.sync_copy(user_hbm.at[pl.ds(ibase, nchunk)], uidx_v)
        pltpu.sync_copy(pos_hbm.at[pl.ds(ibase, nchunk)], pidx_v)
        pltpu.sync_copy(neg_hbm.at[pl.ds(ibase, nchunk)], nidx_v)

        def chunk(j, carry):
            copies = []
            for d in range(_D):
                dsl = pl.ds(d * _STRIDE, _STRIDE)
                copies.append(pltpu.async_copy(
                    utab_hbm.at[dsl].at[uidx_v.at[j]], ubuf_v.at[d], sem))
                copies.append(pltpu.async_copy(
                    itab_hbm.at[dsl].at[pidx_v.at[j]], pbuf_v.at[d], sem))
                copies.append(pltpu.async_copy(
                    itab_hbm.at[dsl].at[nidx_v.at[j]], nbuf_v.at[d], sem))
            for c in copies:
                c.wait()

            for k in range(_CHUNK // 16):
                sl = pl.ds(k * 16, 16)
                accp = jnp.zeros((16,), jnp.float32)
                accn = jnp.zeros((16,), jnp.float32)
                for d in range(_D):
                    u = ubuf_v[d, sl]
                    accp = accp + u * pbuf_v[d, sl]
                    accn = accn + u * nbuf_v[d, sl]
                posbuf_v[pl.ds(j * _CHUNK + k * 16, 16)] = accp
                negbuf_v[pl.ds(j * _CHUNK + k * 16, 16)] = accn
            return carry

        lax.fori_loop(0, nchunk, chunk, 0)

        obase = wid * bpw
        pltpu.sync_copy(posbuf_v, pos_out.at[pl.ds(obase, bpw)])
        pltpu.sync_copy(negbuf_v, neg_out.at[pl.ds(obase, bpw)])

    call = pl.kernel(
        body,
        out_type=(jax.ShapeDtypeStruct((_B,), jnp.float32),
                  jax.ShapeDtypeStruct((_B,), jnp.float32)),
        mesh=mesh,
        scratch_types=[
            pltpu.VMEM((_B // _CHUNK // 32, _CHUNK), jnp.int32),
            pltpu.VMEM((_B // _CHUNK // 32, _CHUNK), jnp.int32),
            pltpu.VMEM((_B // _CHUNK // 32, _CHUNK), jnp.int32),
            pltpu.VMEM((_D, _CHUNK), jnp.float32),
            pltpu.VMEM((_D, _CHUNK), jnp.float32),
            pltpu.VMEM((_D, _CHUNK), jnp.float32),
            pltpu.VMEM((bpw,), jnp.float32),
            pltpu.VMEM((bpw,), jnp.float32),
            pltpu.SemaphoreType.DMA,
        ],
        compiler_params=pltpu.CompilerParams(
            needs_layout_passes=False, use_tc_tiling_on_sc=False),
    )
    return call


def kernel(user, pos_item, neg_item, user_table, item_table):
    call = _build_sc_call()
    u2 = user.astype(jnp.int32).reshape(_B // _CHUNK, _CHUNK)
    p2 = pos_item.astype(jnp.int32).reshape(_B // _CHUNK, _CHUNK)
    n2 = neg_item.astype(jnp.int32).reshape(_B // _CHUNK, _CHUNK)
    ut = jnp.pad(jnp.swapaxes(user_table, 0, 1),
                 ((0, 0), (0, _STRIDE - _NROW))).reshape(_D * _STRIDE)
    it = jnp.pad(jnp.swapaxes(item_table, 0, 1),
                 ((0, 0), (0, _STRIDE - _NROW))).reshape(_D * _STRIDE)
    return call(u2, p2, n2, ut, it)


# speedup vs baseline: 1.0272x; 1.0272x over previous
import jax
import jax.numpy as jnp
from jax import lax
from jax.experimental import pallas as pl
from jax.experimental.pallas import tpu as pltpu, tpu_sc as plsc

_B = 16384
_D = 32
_CHUNK = 128
_NROW = 1000001
_STRIDE = 1000008


def _build_sc_call():
    info = plsc.get_sparse_core_info()
    nc, ns = info.num_cores, info.num_subcores
    nw = nc * ns
    bpw = _B // nw
    nchunk = bpw // _CHUNK

    mesh = plsc.VectorSubcoreMesh(core_axis_name="c", subcore_axis_name="s")

    def body(user_hbm, pos_hbm, neg_hbm, utab_hbm, itab_hbm,
             pos_out, neg_out,
             uidx_v, pidx_v, nidx_v, ubidx_v, pbidx_v, nbidx_v,
             ubuf_v, pbuf_v, nbuf_v,
             posbuf_v, negbuf_v, sem):
        wid = lax.axis_index("s") * nc + lax.axis_index("c")
        ibase = wid * nchunk

        pltpu.sync_copy(user_hbm.at[pl.ds(ibase, nchunk)], uidx_v)
        pltpu.sync_copy(pos_hbm.at[pl.ds(ibase, nchunk)], pidx_v)
        pltpu.sync_copy(neg_hbm.at[pl.ds(ibase, nchunk)], nidx_v)

        def chunk(j, carry):
            for d in range(_D):
                for k in range(_CHUNK // 16):
                    sl = pl.ds(k * 16, 16)
                    ubidx_v[d, sl] = uidx_v[j, sl] + d * _NROW
                    pbidx_v[d, sl] = pidx_v[j, sl] + d * _NROW
                    nbidx_v[d, sl] = nidx_v[j, sl] + d * _NROW
            copies = []
            for d in range(_D):
                copies.append(pltpu.async_copy(
                    utab_hbm.at[ubidx_v.at[d]], ubuf_v.at[d], sem))
                copies.append(pltpu.async_copy(
                    itab_hbm.at[pbidx_v.at[d]], pbuf_v.at[d], sem))
                copies.append(pltpu.async_copy(
                    itab_hbm.at[nbidx_v.at[d]], nbuf_v.at[d], sem))
            for c in copies:
                c.wait()

            for k in range(_CHUNK // 16):
                sl = pl.ds(k * 16, 16)
                accp = jnp.zeros((16,), jnp.float32)
                accn = jnp.zeros((16,), jnp.float32)
                for d in range(_D):
                    u = ubuf_v[d, sl]
                    accp = accp + u * pbuf_v[d, sl]
                    accn = accn + u * nbuf_v[d, sl]
                posbuf_v[pl.ds(j * _CHUNK + k * 16, 16)] = accp
                negbuf_v[pl.ds(j * _CHUNK + k * 16, 16)] = accn
            return carry

        lax.fori_loop(0, nchunk, chunk, 0)

        obase = wid * bpw
        pltpu.sync_copy(posbuf_v, pos_out.at[pl.ds(obase, bpw)])
        pltpu.sync_copy(negbuf_v, neg_out.at[pl.ds(obase, bpw)])

    call = pl.kernel(
        body,
        out_type=(jax.ShapeDtypeStruct((_B,), jnp.float32),
                  jax.ShapeDtypeStruct((_B,), jnp.float32)),
        mesh=mesh,
        scratch_types=[
            pltpu.VMEM((_B // _CHUNK // 32, _CHUNK), jnp.int32),
            pltpu.VMEM((_B // _CHUNK // 32, _CHUNK), jnp.int32),
            pltpu.VMEM((_B // _CHUNK // 32, _CHUNK), jnp.int32),
            pltpu.VMEM((_D, _CHUNK), jnp.int32),
            pltpu.VMEM((_D, _CHUNK), jnp.int32),
            pltpu.VMEM((_D, _CHUNK), jnp.int32),
            pltpu.VMEM((_D, _CHUNK), jnp.float32),
            pltpu.VMEM((_D, _CHUNK), jnp.float32),
            pltpu.VMEM((_D, _CHUNK), jnp.float32),
            pltpu.VMEM((bpw,), jnp.float32),
            pltpu.VMEM((bpw,), jnp.float32),
            pltpu.SemaphoreType.DMA,
        ],
        compiler_params=pltpu.CompilerParams(
            needs_layout_passes=False, use_tc_tiling_on_sc=False),
    )
    return call


def kernel(user, pos_item, neg_item, user_table, item_table):
    call = _build_sc_call()
    u2 = user.astype(jnp.int32).reshape(_B // _CHUNK, _CHUNK)
    p2 = pos_item.astype(jnp.int32).reshape(_B // _CHUNK, _CHUNK)
    n2 = neg_item.astype(jnp.int32).reshape(_B // _CHUNK, _CHUNK)
    ut = jnp.swapaxes(user_table, 0, 1).reshape(_D * _NROW)
    it = jnp.swapaxes(item_table, 0, 1).reshape(_D * _NROW)
    return call(u2, p2, n2, ut, it)


# speedup vs baseline: 5.7818x; 5.6287x over previous
import jax
import jax.numpy as jnp
from jax import lax
from jax.experimental import pallas as pl
from jax.experimental.pallas import tpu as pltpu, tpu_sc as plsc

_B = 16384
_D = 32
_IDX_CHUNK = 128


def _build_sc_call():
    info = plsc.get_sparse_core_info()
    nc, ns = info.num_cores, info.num_subcores
    nw = nc * ns
    bpw = _B // nw
    nchunk = bpw // _IDX_CHUNK
    idx_rows_per_w = bpw // _IDX_CHUNK

    mesh = plsc.VectorSubcoreMesh(core_axis_name="c", subcore_axis_name="s")

    def body(user_hbm, pos_hbm, neg_hbm, utab_hbm, itab_hbm,
             pos_out, neg_out,
             uidx_v, pidx_v, nidx_v, urows_v, prows_v, nrows_v,
             posbuf_v, negbuf_v, sem):
        wid = lax.axis_index("s") * nc + lax.axis_index("c")
        ibase = wid * idx_rows_per_w

        pltpu.sync_copy(user_hbm.at[pl.ds(ibase, nchunk)], uidx_v)
        pltpu.sync_copy(pos_hbm.at[pl.ds(ibase, nchunk)], pidx_v)
        pltpu.sync_copy(neg_hbm.at[pl.ds(ibase, nchunk)], nidx_v)

        copies = []
        for j in range(nchunk):
            dst = pl.ds(j * _IDX_CHUNK, _IDX_CHUNK)
            copies.append(pltpu.async_copy(
                utab_hbm.at[uidx_v.at[j]], urows_v.at[dst], sem))
            copies.append(pltpu.async_copy(
                itab_hbm.at[pidx_v.at[j]], prows_v.at[dst], sem))
            copies.append(pltpu.async_copy(
                itab_hbm.at[nidx_v.at[j]], nrows_v.at[dst], sem))
        for c in copies:
            c.wait()

        lanes = lax.iota(jnp.int32, 16)

        def group(g, carry):
            row0 = g * 16
            rows = row0 + lanes
            accp = jnp.zeros((16,), jnp.float32)
            accn = jnp.zeros((16,), jnp.float32)
            for dd in range(_D):
                cols = jnp.full((16,), dd, jnp.int32)
                u = plsc.load_gather(urows_v, [rows, cols])
                p = plsc.load_gather(prows_v, [rows, cols])
                n = plsc.load_gather(nrows_v, [rows, cols])
                accp = accp + u * p
                accn = accn + u * n
            posbuf_v[pl.ds(row0, 16)] = accp
            negbuf_v[pl.ds(row0, 16)] = accn
            return carry

        lax.fori_loop(0, bpw // 16, group, 0)

        obase = wid * bpw
        pltpu.sync_copy(posbuf_v, pos_out.at[pl.ds(obase, bpw)])
        pltpu.sync_copy(negbuf_v, neg_out.at[pl.ds(obase, bpw)])

    call = pl.kernel(
        body,
        out_type=(jax.ShapeDtypeStruct((_B,), jnp.float32),
                  jax.ShapeDtypeStruct((_B,), jnp.float32)),
        mesh=mesh,
        scratch_types=[
            pltpu.VMEM((nchunk, _IDX_CHUNK), jnp.int32),
            pltpu.VMEM((nchunk, _IDX_CHUNK), jnp.int32),
            pltpu.VMEM((nchunk, _IDX_CHUNK), jnp.int32),
            pltpu.VMEM((bpw, _D), jnp.float32),
            pltpu.VMEM((bpw, _D), jnp.float32),
            pltpu.VMEM((bpw, _D), jnp.float32),
            pltpu.VMEM((bpw,), jnp.float32),
            pltpu.VMEM((bpw,), jnp.float32),
            pltpu.SemaphoreType.DMA,
        ],
        compiler_params=pltpu.CompilerParams(
            needs_layout_passes=False, use_tc_tiling_on_sc=False),
    )
    return call


def kernel(user, pos_item, neg_item, user_table, item_table):
    call = _build_sc_call()
    u2 = user.astype(jnp.int32).reshape(_B // _IDX_CHUNK, _IDX_CHUNK)
    p2 = pos_item.astype(jnp.int32).reshape(_B // _IDX_CHUNK, _IDX_CHUNK)
    n2 = neg_item.astype(jnp.int32).reshape(_B // _IDX_CHUNK, _IDX_CHUNK)
    return call(u2, p2, n2, user_table, item_table)
